# baseline (speedup 1.0000x reference)
import functools

import jax
import jax.numpy as jnp
from jax import lax
from jax.experimental import pallas as pl
from jax.experimental.pallas import tpu as pltpu
from jax.experimental.pallas import tpu_sc as plsc

MP, MM, FF, BB = 8192, 16384, 64, 16
NNZ = 134217
NC, NS, LANES = 2, 16, 16
CH = 128
NB = 3
NB2 = 3
NNZP = ((NNZ + NS * CH * NB - 1) // (NS * CH * NB)) * (NS * CH * NB)
NNZ_PER_TILE = NNZP // NS
NCH = NNZ_PER_TILE // CH
LAST_FULL = NNZ // NNZ_PER_TILE
TAIL_VALID = NNZ - LAST_FULL * NNZ_PER_TILE
TAIL_Z0 = (TAIL_VALID // LANES) * LANES
B_PER_CORE = BB // NC
RPT = MP // NS
FV = FF // LANES
ZR = 32


def _sc_body(x_hbm, rows_hbm, cols_hbm, vals_hbm, out_hbm,
             rows_v, cabs_v, vals_v, gbuf, sbuf, zeros_v, acc_sh,
             gsem, ssem):
    cid = lax.axis_index("c")
    sid = lax.axis_index("s")

    def zloop(i, _):
        for f in range(FV):
            zeros_v[i, pl.ds(f * LANES, LANES)] = jnp.zeros((LANES,), jnp.float32)
        return 0
    lax.fori_loop(0, ZR, zloop, 0)

    izero = jnp.zeros((LANES,), jnp.int32)
    fzero = jnp.zeros((LANES,), jnp.float32)

    @pl.when(sid == LAST_FULL)
    def _():
        def ztail(i, _):
            sl = pl.ds(TAIL_Z0 + i * LANES, LANES)
            rows_v[sl] = izero
            cabs_v[sl] = izero
            vals_v[sl] = fzero
            return 0
        lax.fori_loop(0, (NNZ_PER_TILE - TAIL_Z0) // LANES, ztail, 0)
        pltpu.sync_copy(rows_hbm.at[pl.ds(sid * NNZ_PER_TILE, TAIL_VALID)],
                        rows_v.at[pl.ds(0, TAIL_VALID)])
        pltpu.sync_copy(cols_hbm.at[pl.ds(sid * NNZ_PER_TILE, TAIL_VALID)],
                        cabs_v.at[pl.ds(0, TAIL_VALID)])
        pltpu.sync_copy(vals_hbm.at[pl.ds(sid * NNZ_PER_TILE, TAIL_VALID)],
                        vals_v.at[pl.ds(0, TAIL_VALID)])

    @pl.when(sid != LAST_FULL)
    def _():
        pltpu.sync_copy(rows_hbm.at[pl.ds(sid * NNZ_PER_TILE, NNZ_PER_TILE)],
                        rows_v)
        pltpu.sync_copy(cols_hbm.at[pl.ds(sid * NNZ_PER_TILE, NNZ_PER_TILE)],
                        cabs_v)
        pltpu.sync_copy(vals_hbm.at[pl.ds(sid * NNZ_PER_TILE, NNZ_PER_TILE)],
                        vals_v)

    for r in range(RPT // ZR):
        pltpu.sync_copy(zeros_v,
                        acc_sh.at[pl.ds(sid * RPT + r * ZR, ZR)])

    def cadd(base):
        def cloop(i, _):
            sl = pl.ds(i * LANES, LANES)
            cabs_v[sl] = cabs_v[sl] + base
            return 0
        lax.fori_loop(0, NNZ_PER_TILE // LANES, cloop, 0)

    cadd(cid * B_PER_CORE * MM)
    plsc.subcore_barrier()

    def batch_body(bi, _):
        b = cid * B_PER_CORE + bi

        for t in range(NB):
            pltpu.async_copy(x_hbm.at[cabs_v.at[pl.ds(t * CH, CH)]],
                             gbuf.at[t], gsem.at[t])

        def round_body(jr, _):
            for u in range(NB):
                j = jr * NB + u
                us = u % NB2

                pltpu.make_async_copy(
                    x_hbm.at[cabs_v.at[pl.ds(j * CH, CH)]],
                    gbuf.at[u], gsem.at[u]).wait()

                @pl.when(j >= NB2)
                def _():
                    pltpu.make_async_copy(
                        sbuf.at[us], acc_sh.at[rows_v.at[pl.ds(j * CH, CH)]],
                        ssem.at[us]).wait()

                def scale_body(g, _):
                    v16 = vals_v[pl.ds(j * CH + g * LANES, LANES)]
                    for k in range(LANES):
                        bv = lax.gather(
                            v16, jnp.full((LANES, 1), k, jnp.int32),
                            lax.GatherDimensionNumbers(
                                offset_dims=(), collapsed_slice_dims=(0,),
                                start_index_map=(0,)),
                            (1,),
                            mode=lax.GatherScatterMode.PROMISE_IN_BOUNDS)
                        i = g * LANES + k
                        for f in range(FV):
                            sl = pl.ds(f * LANES, LANES)
                            sbuf[us, i, sl] = gbuf[u, i, sl] * bv
                    return 0
                lax.fori_loop(0, CH // LANES, scale_body, 0)

                pltpu.async_copy(sbuf.at[us],
                                 acc_sh.at[rows_v.at[pl.ds(j * CH, CH)]],
                                 ssem.at[us], add=True)

                @pl.when(j + NB < NCH)
                def _():
                    pltpu.async_copy(
                        x_hbm.at[cabs_v.at[pl.ds((j + NB) * CH, CH)]],
                        gbuf.at[u], gsem.at[u])
            return 0
        lax.fori_loop(0, NCH // NB, round_body, 0)

        for t in range(NB2):
            j2 = NCH - NB2 + t
            pltpu.make_async_copy(
                sbuf.at[j2 % NB2], acc_sh.at[rows_v.at[pl.ds(j2 * CH, CH)]],
                ssem.at[j2 % NB2]).wait()

        plsc.subcore_barrier()

        pltpu.sync_copy(acc_sh.at[pl.ds(sid * RPT, RPT)],
                        out_hbm.at[pl.ds(b * MP + sid * RPT, RPT)])

        @pl.when(bi + 1 < B_PER_CORE)
        def _():
            for r in range(RPT // ZR):
                pltpu.sync_copy(
                    zeros_v, acc_sh.at[pl.ds(sid * RPT + r * ZR, ZR)])
            cadd(MM)

        plsc.subcore_barrier()
        return 0

    lax.fori_loop(0, B_PER_CORE, batch_body, 0)


def kernel(x, vals, rows, cols):
    x2d = x.reshape(BB * MM, FF)

    mesh = plsc.VectorSubcoreMesh(
        core_axis_name="c", subcore_axis_name="s",
        num_cores=NC, num_subcores=NS)

    f = functools.partial(
        pl.kernel,
        out_type=jax.ShapeDtypeStruct((BB * MP, FF), jnp.float32),
        mesh=mesh,
        compiler_params=pltpu.CompilerParams(use_tc_tiling_on_sc=False),
        scratch_types=[
            pltpu.VMEM((NNZ_PER_TILE,), jnp.int32),
            pltpu.VMEM((NNZ_PER_TILE,), jnp.int32),
            pltpu.VMEM((NNZ_PER_TILE,), jnp.float32),
            pltpu.VMEM((NB, CH, FF), jnp.float32),
            pltpu.VMEM((NB2, CH, FF), jnp.float32),
            pltpu.VMEM((ZR, FF), jnp.float32),
            pltpu.VMEM_SHARED((MP, FF), jnp.float32),
            pltpu.SemaphoreType.DMA((NB,)),
            pltpu.SemaphoreType.DMA((NB2,)),
        ],
    )(_sc_body)

    out2d = f(x2d, rows, cols, vals)
    return out2d.reshape(BB, MP, FF)


# speedup vs baseline: 1.0119x; 1.0119x over previous
import functools

import jax
import jax.numpy as jnp
from jax import lax
from jax.experimental import pallas as pl
from jax.experimental.pallas import tpu as pltpu
from jax.experimental.pallas import tpu_sc as plsc

MP, MM, FF, BB = 8192, 16384, 64, 16
NNZ = 134217
NC, NS, LANES = 2, 16, 16
CH = 128
NB = 3
NB2 = 3
NNZP = ((NNZ + NS * CH * NB - 1) // (NS * CH * NB)) * (NS * CH * NB)
NNZ_PER_TILE = NNZP // NS
NCH = NNZ_PER_TILE // CH
LAST_FULL = NNZ // NNZ_PER_TILE
TAIL_VALID = NNZ - LAST_FULL * NNZ_PER_TILE
TAIL_Z0 = (TAIL_VALID // LANES) * LANES
B_PER_CORE = BB // NC
RPT = MP // NS
FV = FF // LANES
ZR = 32


def _sc_body(x_hbm, rows_hbm, cols_hbm, vals_hbm, out_hbm,
             rows_v, cabs_v, vals_v, gbuf, sbuf, zeros_v, acc_sh,
             gsem, ssem):
    cid = lax.axis_index("c")
    sid = lax.axis_index("s")

    def zloop(i, _):
        for f in range(FV):
            zeros_v[i, pl.ds(f * LANES, LANES)] = jnp.zeros((LANES,), jnp.float32)
        return 0
    lax.fori_loop(0, ZR, zloop, 0)

    izero = jnp.zeros((LANES,), jnp.int32)
    fzero = jnp.zeros((LANES,), jnp.float32)

    @pl.when(sid == LAST_FULL)
    def _():
        def ztail(i, _):
            sl = pl.ds(TAIL_Z0 + i * LANES, LANES)
            rows_v[sl] = izero
            cabs_v[sl] = izero
            vals_v[sl] = fzero
            return 0
        lax.fori_loop(0, (NNZ_PER_TILE - TAIL_Z0) // LANES, ztail, 0)
        pltpu.sync_copy(rows_hbm.at[pl.ds(sid * NNZ_PER_TILE, TAIL_VALID)],
                        rows_v.at[pl.ds(0, TAIL_VALID)])
        pltpu.sync_copy(cols_hbm.at[pl.ds(sid * NNZ_PER_TILE, TAIL_VALID)],
                        cabs_v.at[pl.ds(0, TAIL_VALID)])
        pltpu.sync_copy(vals_hbm.at[pl.ds(sid * NNZ_PER_TILE, TAIL_VALID)],
                        vals_v.at[pl.ds(0, TAIL_VALID)])

    @pl.when(sid != LAST_FULL)
    def _():
        pltpu.sync_copy(rows_hbm.at[pl.ds(sid * NNZ_PER_TILE, NNZ_PER_TILE)],
                        rows_v)
        pltpu.sync_copy(cols_hbm.at[pl.ds(sid * NNZ_PER_TILE, NNZ_PER_TILE)],
                        cabs_v)
        pltpu.sync_copy(vals_hbm.at[pl.ds(sid * NNZ_PER_TILE, NNZ_PER_TILE)],
                        vals_v)

    for r in range(RPT // ZR):
        pltpu.sync_copy(zeros_v,
                        acc_sh.at[pl.ds(sid * RPT + r * ZR, ZR)])

    def cadd(base):
        def cloop(i, _):
            sl = pl.ds(i * LANES, LANES)
            cabs_v[sl] = cabs_v[sl] + base
            return 0
        lax.fori_loop(0, NNZ_PER_TILE // LANES, cloop, 0)

    cadd(cid * B_PER_CORE * MM)
    plsc.subcore_barrier()

    for t in range(NB):
        pltpu.async_copy(x_hbm.at[cabs_v.at[pl.ds(t * CH, CH)]],
                         gbuf.at[t], gsem.at[t])

    def batch_body(bi, _):
        b = cid * B_PER_CORE + bi

        def round_body(jr, _):
            for u in range(NB):
                j = jr * NB + u
                us = u % NB2

                pltpu.make_async_copy(
                    x_hbm.at[cabs_v.at[pl.ds(j * CH, CH)]],
                    gbuf.at[u], gsem.at[u]).wait()

                @pl.when(j >= NB2)
                def _():
                    pltpu.make_async_copy(
                        sbuf.at[us], acc_sh.at[rows_v.at[pl.ds(j * CH, CH)]],
                        ssem.at[us]).wait()

                def scale_body(g, _):
                    v16 = vals_v[pl.ds(j * CH + g * LANES, LANES)]
                    for k in range(LANES):
                        bv = lax.gather(
                            v16, jnp.full((LANES, 1), k, jnp.int32),
                            lax.GatherDimensionNumbers(
                                offset_dims=(), collapsed_slice_dims=(0,),
                                start_index_map=(0,)),
                            (1,),
                            mode=lax.GatherScatterMode.PROMISE_IN_BOUNDS)
                        i = g * LANES + k
                        for f in range(FV):
                            sl = pl.ds(f * LANES, LANES)
                            sbuf[us, i, sl] = gbuf[u, i, sl] * bv
                    return 0
                lax.fori_loop(0, CH // LANES, scale_body, 0)

                pltpu.async_copy(sbuf.at[us],
                                 acc_sh.at[rows_v.at[pl.ds(j * CH, CH)]],
                                 ssem.at[us], add=True)

                @pl.when(j + NB < NCH)
                def _():
                    pltpu.async_copy(
                        x_hbm.at[cabs_v.at[pl.ds((j + NB) * CH, CH)]],
                        gbuf.at[u], gsem.at[u])
            return 0
        lax.fori_loop(0, NCH // NB, round_body, 0)

        for t in range(NB2):
            j2 = NCH - NB2 + t
            pltpu.make_async_copy(
                sbuf.at[j2 % NB2], acc_sh.at[rows_v.at[pl.ds(j2 * CH, CH)]],
                ssem.at[j2 % NB2]).wait()

        plsc.subcore_barrier()

        @pl.when(bi + 1 < B_PER_CORE)
        def _():
            cadd(MM)
            for t in range(NB):
                pltpu.async_copy(x_hbm.at[cabs_v.at[pl.ds(t * CH, CH)]],
                                 gbuf.at[t], gsem.at[t])

        pltpu.sync_copy(acc_sh.at[pl.ds(sid * RPT, RPT)],
                        out_hbm.at[pl.ds(b * MP + sid * RPT, RPT)])

        @pl.when(bi + 1 < B_PER_CORE)
        def _():
            for r in range(RPT // ZR):
                pltpu.sync_copy(
                    zeros_v, acc_sh.at[pl.ds(sid * RPT + r * ZR, ZR)])

        plsc.subcore_barrier()
        return 0

    lax.fori_loop(0, B_PER_CORE, batch_body, 0)


def kernel(x, vals, rows, cols):
    x2d = x.reshape(BB * MM, FF)

    mesh = plsc.VectorSubcoreMesh(
        core_axis_name="c", subcore_axis_name="s",
        num_cores=NC, num_subcores=NS)

    f = functools.partial(
        pl.kernel,
        out_type=jax.ShapeDtypeStruct((BB * MP, FF), jnp.float32),
        mesh=mesh,
        compiler_params=pltpu.CompilerParams(use_tc_tiling_on_sc=False),
        scratch_types=[
            pltpu.VMEM((NNZ_PER_TILE,), jnp.int32),
            pltpu.VMEM((NNZ_PER_TILE,), jnp.int32),
            pltpu.VMEM((NNZ_PER_TILE,), jnp.float32),
            pltpu.VMEM((NB, CH, FF), jnp.float32),
            pltpu.VMEM((NB2, CH, FF), jnp.float32),
            pltpu.VMEM((ZR, FF), jnp.float32),
            pltpu.VMEM_SHARED((MP, FF), jnp.float32),
            pltpu.SemaphoreType.DMA((NB,)),
            pltpu.SemaphoreType.DMA((NB2,)),
        ],
    )(_sc_body)

    out2d = f(x2d, rows, cols, vals)
    return out2d.reshape(BB, MP, FF)
